# Initial kernel scaffold; baseline (speedup 1.0000x reference)
import jax
import jax.numpy as jnp
from jax.experimental import pallas as pl


def kernel(x_nchw, w1r, b1p, w2r, b2p, fc1_w, fc1_b, fc2_w, fc2_b):
    raise NotImplementedError("write your pallas kernel here")


# speedup vs baseline: 47.7537x; 47.7537x over previous
import functools

import jax
import jax.numpy as jnp
from jax.experimental import pallas as pl
from jax.experimental.pallas import tpu as pltpu

_H = 128


def _fused_kernel(x_ref, w1c_ref, b1_ref, w2c_ref, b2_ref,
                  fc1w_ref, fc1b_ref, fc2w_ref, fc2b_ref, o_ref, *, bb):
    x = x_ref[...]

    xc = jnp.concatenate([x[:, i:i + 24, :] for i in range(5)], axis=-1)
    acc = jnp.dot(xc.reshape(bb * 24, 5 * 28), w1c_ref[...],
                  preferred_element_type=jnp.float32)
    wp = jnp.maximum(acc[:, :_H], acc[:, _H:]).reshape(bb, 12, 2 * _H)
    y1 = jnp.maximum(wp[:, :, :_H], wp[:, :, _H:])
    y1 = jnp.maximum(y1 + b1_ref[...], 0.0)

    yc = jnp.concatenate([y1[:, i:i + 8, :] for i in range(5)], axis=-1)
    acc2 = jnp.dot(yc.reshape(bb * 8, 5 * _H), w2c_ref[...],
                   preferred_element_type=jnp.float32)
    wp2 = jnp.maximum(acc2[:, :_H], acc2[:, _H:]).reshape(bb, 4, 2 * _H)
    y2 = jnp.maximum(wp2[:, :, :_H], wp2[:, :, _H:])
    y2 = jnp.maximum(y2 + b2_ref[...], 0.0)

    a = y2.reshape(bb, 4 * _H)
    h = jnp.dot(a, fc1w_ref[...], preferred_element_type=jnp.float32)
    h = jnp.maximum(h + fc1b_ref[...], 0.0)
    z = jnp.dot(h, fc2w_ref[...], preferred_element_type=jnp.float32)
    z = z + fc2b_ref[...]
    s = z - jnp.max(z, axis=-1, keepdims=True)
    o_ref[...] = s - jnp.log(jnp.sum(jnp.exp(s), axis=-1, keepdims=True))


def kernel(x_nchw, w1r, b1p, w2r, b2p, fc1_w, fc1_b, fc2_w, fc2_b):
    B = x_nchw.shape[0]
    x = x_nchw.reshape(B, 28, 28)
    w1c = w1r.reshape(5 * 28, 2 * _H)
    w2c = w2r.reshape(5 * _H, 2 * _H)
    n_out = fc2_w.shape[1]

    bb = next(s for s in (256, 128, 64, 32, 16, 8, 4, 2, 1) if B % s == 0)
    kern = functools.partial(_fused_kernel, bb=bb)
    flops = 2 * B * (24 * 140 * 256 + 8 * 640 * 256 + 512 * 50 + 50 * 10)
    bytes_accessed = 4 * (B * 28 * 28 + B * n_out) + 4 * (w1c.size + w2c.size
                                                          + fc1_w.size)
    return pl.pallas_call(
        kern,
        out_shape=jax.ShapeDtypeStruct((B, n_out), jnp.float32),
        grid=(B // bb,),
        in_specs=[
            pl.BlockSpec((bb, 28, 28), lambda b: (b, 0, 0)),
            pl.BlockSpec((5 * 28, 2 * _H), lambda b: (0, 0)),
            pl.BlockSpec((1, _H), lambda b: (0, 0)),
            pl.BlockSpec((5 * _H, 2 * _H), lambda b: (0, 0)),
            pl.BlockSpec((1, _H), lambda b: (0, 0)),
            pl.BlockSpec((4 * _H, fc1_w.shape[1]), lambda b: (0, 0)),
            pl.BlockSpec((1, fc1_b.shape[1]), lambda b: (0, 0)),
            pl.BlockSpec((fc2_w.shape[0], n_out), lambda b: (0, 0)),
            pl.BlockSpec((1, n_out), lambda b: (0, 0)),
        ],
        out_specs=pl.BlockSpec((bb, n_out), lambda b: (b, 0)),
        compiler_params=pltpu.CompilerParams(dimension_semantics=("parallel",)),
        cost_estimate=pl.CostEstimate(flops=flops, transcendentals=B * 11,
                                      bytes_accessed=bytes_accessed),
    )(x, w1c, b1p, w2c, b2p, fc1_w, fc1_b, fc2_w, fc2_b)
